# Initial kernel scaffold; baseline (speedup 1.0000x reference)
import jax
import jax.numpy as jnp
from jax.experimental import pallas as pl


def kernel(x, edge_index, edge_attr, params):
    raise NotImplementedError("write your pallas kernel here")


# speedup vs baseline: 7.5571x; 7.5571x over previous
import functools

import jax
import jax.numpy as jnp
from jax import lax
from jax.experimental import pallas as pl
from jax.experimental.pallas import tpu as pltpu
from jax.experimental.pallas import tpu_sc as plsc

N = 100000
E = 1600000
D = 32

EPAD = 1638400
NPAD = 38400
ROWS = EPAD // 128
CHUNK = 2048
NACC = 100096
ZTAIL = NACC // 16 - 3 * 2048

_mesh = plsc.VectorSubcoreMesh(core_axis_name="c", subcore_axis_name="s")
_sc_params = pltpu.CompilerParams(needs_layout_passes=False,
                                  use_tc_tiling_on_sc=False)


def _zero16f():
    return jnp.zeros((16,), jnp.float32)


@functools.partial(
    pl.kernel,
    out_type=[jax.ShapeDtypeStruct((NACC,), jnp.float32),
              jax.ShapeDtypeStruct((NACC,), jnp.float32)],
    scratch_types=[
        pltpu.VMEM_SHARED((NACC,), jnp.float32),
        pltpu.VMEM((NACC // 128, 128), jnp.float32),
        pltpu.VMEM((CHUNK,), jnp.float32),
        pltpu.VMEM((16, 128), jnp.int32),
        pltpu.VMEM((16, 128), jnp.int32),
        pltpu.VMEM((16, 128), jnp.float32),
        pltpu.VMEM((16, 128), jnp.float32),
        pltpu.VMEM((2, 16), jnp.float32),
    ],
    mesh=_mesh,
    compiler_params=_sc_params,
)
def _edge1(x_hbm, src_hbm, dst_hbm, a_hbm, wb_hbm,
           out0, out1, acc, x_v, z_v, src_c, dst_c, a_c, m_c, wb_v):
    c = lax.axis_index("c")
    s = lax.axis_index("s")
    wid = s * 2 + c
    pltpu.sync_copy(x_hbm, x_v)
    pltpu.sync_copy(wb_hbm, wb_v)
    z16 = _zero16f()

    def zb(k, carry):
        z_v[pl.ds(k * 16, 16)] = z16
        return carry
    lax.fori_loop(0, CHUNK // 16, zb, 0)
    base = s * (NACC // 16)
    for t in range(3):
        pltpu.sync_copy(z_v, acc.at[pl.ds(base + t * CHUNK, CHUNK)])
    pltpu.sync_copy(z_v.at[pl.ds(0, ZTAIL)], acc.at[pl.ds(base + 3 * CHUNK, ZTAIL)])
    plsc.subcore_barrier()

    wv = wb_v[0]
    bv = wb_v[1]

    def chunk_body(g, carry):
        rb = wid * 400 + g * 16
        pltpu.sync_copy(src_hbm.at[pl.ds(rb, 16)], src_c)
        pltpu.sync_copy(dst_hbm.at[pl.ds(rb, 16)], dst_c)
        pltpu.sync_copy(a_hbm.at[pl.ds(rb, 16)], a_c)

        def row_body(j, carry2):
            def lane_body(l, carry3):
                sv = src_c[j, pl.ds(l * 16, 16)]
                av = a_c[j, pl.ds(l * 16, 16)]
                xg = plsc.load_gather(x_v, [sv >> 7, sv & 127])
                m_c[j, pl.ds(l * 16, 16)] = jnp.maximum(xg + av * wv + bv, 0.0)
                return carry3
            return lax.fori_loop(0, 8, lane_body, carry2)
        lax.fori_loop(0, 16, row_body, 0)
        for j in range(16):
            pltpu.sync_copy(m_c.at[j], acc.at[dst_c.at[j]], add=True)
        return carry
    lax.fori_loop(0, 25, chunk_body, 0)
    plsc.subcore_barrier()

    def wb(dst_hbm_ref):
        for t in range(3):
            sl = pl.ds(base + t * CHUNK, CHUNK)
            pltpu.sync_copy(acc.at[sl], z_v)
            pltpu.sync_copy(z_v, dst_hbm_ref.at[sl])
        sl = pl.ds(base + 3 * CHUNK, ZTAIL)
        pltpu.sync_copy(acc.at[sl], z_v.at[pl.ds(0, ZTAIL)])
        pltpu.sync_copy(z_v.at[pl.ds(0, ZTAIL)], dst_hbm_ref.at[sl])

    @pl.when(c == 0)
    def _():
        wb(out0)

    @pl.when(c == 1)
    def _():
        wb(out1)


C2 = 1024
ZT2 = NACC // 16 - 6 * C2


@functools.partial(
    pl.kernel,
    out_type=[jax.ShapeDtypeStruct((NACC, 16), jnp.float32),
              jax.ShapeDtypeStruct((NACC, 16), jnp.float32)],
    scratch_types=[
        pltpu.VMEM_SHARED((NACC, 16), jnp.float32),
        pltpu.VMEM((C2, 16), jnp.float32),
        pltpu.VMEM((8, 128), jnp.int32),
        pltpu.VMEM((8, 128), jnp.int32),
        pltpu.VMEM((8, 128), jnp.float32),
        pltpu.VMEM((2, 2, 16), jnp.float32),
        pltpu.SemaphoreType.DMA,
    ],
    mesh=_mesh,
    compiler_params=_sc_params,
)
def _edge23(hlo_hbm, hhi_hbm, src_hbm, dst_hbm, a_hbm, wb_hbm,
            out0, out1, acc, rows, src_c, dst_c, a_c, wb_v, sem):
    c = lax.axis_index("c")
    s = lax.axis_index("s")
    pltpu.sync_copy(wb_hbm, wb_v)
    z16 = _zero16f()

    def zb(k, carry):
        rows[k] = z16
        return carry
    lax.fori_loop(0, C2, zb, 0)
    base = s * (NACC // 16)
    for t in range(6):
        pltpu.sync_copy(rows, acc.at[pl.ds(base + t * C2, C2)])
    pltpu.sync_copy(rows.at[pl.ds(0, ZT2)], acc.at[pl.ds(base + 6 * C2, ZT2)])
    plsc.subcore_barrier()

    wv = wb_v[c, 0]
    bv = wb_v[c, 1]

    def chunk_body(g, carry):
        rb = s * 800 + g * 8
        pltpu.sync_copy(src_hbm.at[pl.ds(rb, 8)], src_c)
        pltpu.sync_copy(dst_hbm.at[pl.ds(rb, 8)], dst_c)
        pltpu.sync_copy(a_hbm.at[pl.ds(rb, 8)], a_c)

        @pl.when(c == 0)
        def _():
            hs = [pltpu.async_copy(hlo_hbm.at[src_c.at[j]],
                                   rows.at[pl.ds(j * 128, 128)], sem)
                  for j in range(8)]
            for h in hs:
                h.wait()

        @pl.when(c == 1)
        def _():
            hs = [pltpu.async_copy(hhi_hbm.at[src_c.at[j]],
                                   rows.at[pl.ds(j * 128, 128)], sem)
                  for j in range(8)]
            for h in hs:
                h.wait()

        def edge_body(i, carry2):
            ri = jnp.full((16,), i >> 7, jnp.int32)
            ci = jnp.full((16,), i & 127, jnp.int32)
            av = plsc.load_gather(a_c, [ri, ci])
            rows[i] = jnp.maximum(rows[i] + av * wv + bv, 0.0)
            return carry2
        lax.fori_loop(0, C2, edge_body, 0)
        for j in range(8):
            pltpu.sync_copy(rows.at[pl.ds(j * 128, 128)],
                            acc.at[dst_c.at[j]], add=True)
        return carry
    lax.fori_loop(0, 100, chunk_body, 0)
    plsc.subcore_barrier()

    def wb(dst_hbm_ref):
        for t in range(6):
            sl = pl.ds(base + t * C2, C2)
            pltpu.sync_copy(acc.at[sl], rows)
            pltpu.sync_copy(rows, dst_hbm_ref.at[sl])
        sl = pl.ds(base + 6 * C2, ZT2)
        pltpu.sync_copy(acc.at[sl], rows.at[pl.ds(0, ZT2)])
        pltpu.sync_copy(rows.at[pl.ds(0, ZT2)], dst_hbm_ref.at[sl])

    @pl.when(c == 0)
    def _():
        wb(out0)

    @pl.when(c == 1)
    def _():
        wb(out1)


R = 5000
G = N // R


def _leaky(z):
    return jnp.where(z >= 0, z, 0.01 * z)


def _full_spec(shape):
    return pl.BlockSpec(shape, lambda i: tuple(0 for _ in shape))


def _mlp1_body(x_ref, p0_ref, p1_ref, w1, b1, w2, b2, w3, b3, sc_ref,
               h_ref, st_ref):
    h0 = sc_ref[0, 0] * x_ref[...] + p0_ref[...] + p1_ref[...]
    z = _leaky(h0 * w1[...] + b1[...])
    z = _leaky(jnp.dot(z, w2[...], preferred_element_type=jnp.float32) + b2[...])
    h = jnp.dot(z, w3[...], preferred_element_type=jnp.float32) + b3[...]
    h_ref[...] = h
    st_ref[...] = jnp.stack([jnp.sum(h, 0), jnp.sum(h * h, 0)])[None]


def _mlp23_body(hlo_ref, hhi_ref, alo_ref, ahi_ref, w1, b1, w2, b2, w3, b3,
                sc_ref, h_ref, st_ref):
    h = jnp.concatenate([hlo_ref[...], hhi_ref[...]], axis=1)
    ag = jnp.concatenate([alo_ref[...], ahi_ref[...]], axis=1)
    z = sc_ref[0, 0] * h + ag
    z = _leaky(jnp.dot(z, w1[...], preferred_element_type=jnp.float32) + b1[...])
    z = _leaky(jnp.dot(z, w2[...], preferred_element_type=jnp.float32) + b2[...])
    h = jnp.dot(z, w3[...], preferred_element_type=jnp.float32) + b3[...]
    h_ref[...] = h
    st_ref[...] = jnp.stack([jnp.sum(h, 0), jnp.sum(h * h, 0)])[None]


def _bn_split_body(h_ref, st_ref, g_ref, b_ref, lo_ref, hi_ref):
    st = st_ref[...]
    mu = jnp.sum(st[:, 0, :], 0) / N
    var = jnp.sum(st[:, 1, :], 0) / N - mu * mu
    y = g_ref[...] * (h_ref[...] - mu) * lax.rsqrt(var + 1e-5) + b_ref[...]
    y = jnp.maximum(y, 0.0)
    lo_ref[...] = y[:, :16]
    hi_ref[...] = y[:, 16:]


def _bn3_body(h_ref, st_ref, g_ref, b_ref, o_ref):
    st = st_ref[...]
    mu = jnp.sum(st[:, 0, :], 0) / N
    var = jnp.sum(st[:, 1, :], 0) / N - mu * mu
    o_ref[...] = g_ref[...] * (h_ref[...] - mu) * lax.rsqrt(var + 1e-5) + b_ref[...]


def _mlp1(x2, p0, p1, w1, b1, w2, b2, w3, b3, sc):
    return pl.pallas_call(
        _mlp1_body,
        grid=(G,),
        in_specs=[
            pl.BlockSpec((R, 1), lambda i: (i, 0)),
            pl.BlockSpec((R, 1), lambda i: (i, 0)),
            pl.BlockSpec((R, 1), lambda i: (i, 0)),
            _full_spec((1, 32)), _full_spec((1, 32)),
            _full_spec((32, 32)), _full_spec((1, 32)),
            _full_spec((32, 32)), _full_spec((1, 32)),
            _full_spec((1, 1)),
        ],
        out_specs=[
            pl.BlockSpec((R, 32), lambda i: (i, 0)),
            pl.BlockSpec((1, 2, 32), lambda i: (i, 0, 0)),
        ],
        out_shape=[
            jax.ShapeDtypeStruct((N, 32), jnp.float32),
            jax.ShapeDtypeStruct((G, 2, 32), jnp.float32),
        ],
    )(x2, p0, p1, w1, b1, w2, b2, w3, b3, sc)


def _mlp23(hlo, hhi, alo, ahi, w1, b1, w2, b2, w3, b3, sc, dout):
    return pl.pallas_call(
        _mlp23_body,
        grid=(G,),
        in_specs=[
            pl.BlockSpec((R, 16), lambda i: (i, 0)),
            pl.BlockSpec((R, 16), lambda i: (i, 0)),
            pl.BlockSpec((R, 16), lambda i: (i, 0)),
            pl.BlockSpec((R, 16), lambda i: (i, 0)),
            _full_spec((32, 32)), _full_spec((1, 32)),
            _full_spec((32, 32)), _full_spec((1, 32)),
            _full_spec((32, dout)), _full_spec((1, dout)),
            _full_spec((1, 1)),
        ],
        out_specs=[
            pl.BlockSpec((R, dout), lambda i: (i, 0)),
            pl.BlockSpec((1, 2, dout), lambda i: (i, 0, 0)),
        ],
        out_shape=[
            jax.ShapeDtypeStruct((N, dout), jnp.float32),
            jax.ShapeDtypeStruct((G, 2, dout), jnp.float32),
        ],
    )(hlo, hhi, alo, ahi, w1, b1, w2, b2, w3, b3, sc)


def _bn_split(h, st, g, b):
    return pl.pallas_call(
        _bn_split_body,
        grid=(G,),
        in_specs=[
            pl.BlockSpec((R, 32), lambda i: (i, 0)),
            _full_spec((G, 2, 32)),
            _full_spec((1, 32)), _full_spec((1, 32)),
        ],
        out_specs=[
            pl.BlockSpec((R, 16), lambda i: (i, 0)),
            pl.BlockSpec((R, 16), lambda i: (i, 0)),
        ],
        out_shape=[
            jax.ShapeDtypeStruct((N, 16), jnp.float32),
            jax.ShapeDtypeStruct((N, 16), jnp.float32),
        ],
    )(h, st, g, b)


def _bn3(h, st, g, b):
    return pl.pallas_call(
        _bn3_body,
        grid=(G,),
        in_specs=[
            pl.BlockSpec((R, 1), lambda i: (i, 0)),
            _full_spec((G, 2, 1)),
            _full_spec((1, 1)), _full_spec((1, 1)),
        ],
        out_specs=pl.BlockSpec((R, 1), lambda i: (i, 0)),
        out_shape=jax.ShapeDtypeStruct((N, 1), jnp.float32),
    )(h, st, g, b)


def kernel(x, edge_index, edge_attr, params):
    src = edge_index[0]
    dst = edge_index[1]
    a = edge_attr[:, 0]
    pad_src = (jnp.arange(NPAD, dtype=jnp.int32) * 911) % N
    src2 = jnp.concatenate([src, pad_src]).reshape(ROWS, 128)
    dst2 = jnp.concatenate([dst, jnp.full((NPAD,), N, jnp.int32)]).reshape(ROWS, 128)
    a2 = jnp.concatenate([a, jnp.zeros((NPAD,), jnp.float32)]).reshape(ROWS, 128)

    p = params
    wb1 = jnp.stack([jnp.full((16,), p['We1'][0, 0]), jnp.full((16,), p['be1'][0])])
    wb2 = jnp.stack([p['We2'][0].reshape(2, 16), p['be2'].reshape(2, 16)], axis=1)
    wb3 = jnp.stack([p['We3'][0].reshape(2, 16), p['be3'].reshape(2, 16)], axis=1)

    def lin(layers, i):
        W, b = layers[i]
        return W, b.reshape(1, -1)

    sc1 = (1.0 + p['eps1']).reshape(1, 1)
    sc2 = (1.0 + p['eps2']).reshape(1, 1)
    sc3 = (1.0 + p['eps3']).reshape(1, 1)

    x2d = jnp.concatenate([x, jnp.zeros((NACC - N,), jnp.float32)]).reshape(NACC // 128, 128)
    p0, p1 = _edge1(x2d, src2, dst2, a2, wb1)
    w1, b1 = lin(p['nn1'], 0)
    w2, b2 = lin(p['nn1'], 1)
    w3, b3 = lin(p['nn1'], 2)
    h1, st1 = _mlp1(x.reshape(N, 1), p0[:N].reshape(N, 1), p1[:N].reshape(N, 1),
                    w1, b1, w2, b2, w3, b3, sc1)
    lo1, hi1 = _bn_split(h1, st1, p['bn1_g'].reshape(1, 32), p['bn1_b'].reshape(1, 32))

    alo2, ahi2 = _edge23(lo1, hi1, src2, dst2, a2, wb2)
    w1, b1 = lin(p['nn2'], 0)
    w2, b2 = lin(p['nn2'], 1)
    w3, b3 = lin(p['nn2'], 2)
    h3, st2 = _mlp23(lo1, hi1, alo2[:N], ahi2[:N],
                     w1, b1, w2, b2, w3, b3, sc2, 32)
    lo2, hi2 = _bn_split(h3, st2, p['bn2_g'].reshape(1, 32), p['bn2_b'].reshape(1, 32))

    alo3, ahi3 = _edge23(lo2, hi2, src2, dst2, a2, wb3)
    w1, b1 = lin(p['nn3'], 0)
    w2, b2 = lin(p['nn3'], 1)
    w3, b3 = lin(p['nn3'], 2)
    h5, st3 = _mlp23(lo2, hi2, alo3[:N], ahi3[:N],
                     w1, b1, w2, b2, w3, b3, sc3, 1)
    out = _bn3(h5, st3, p['bn3_g'].reshape(1, 1), p['bn3_b'].reshape(1, 1))
    return out


# speedup vs baseline: 10.1975x; 1.3494x over previous
import functools

import jax
import jax.numpy as jnp
from jax import lax
from jax.experimental import pallas as pl
from jax.experimental.pallas import tpu as pltpu
from jax.experimental.pallas import tpu_sc as plsc

N = 100000
E = 1600000
D = 32

EPAD = 1622016
NPAD = EPAD - E
ROWS = EPAD // 128
NACC = 100096
NTILE = NACC // 16

C1 = 4224
R1T = EPAD // 32 // 128
C2 = 512
NC2 = EPAD // 16 // C2

_mesh = plsc.VectorSubcoreMesh(core_axis_name="c", subcore_axis_name="s")
_sc_params = pltpu.CompilerParams(needs_layout_passes=False,
                                  use_tc_tiling_on_sc=False)

_i32 = jnp.int32
_f32 = jnp.float32


@functools.partial(
    pl.kernel,
    out_type=[jax.ShapeDtypeStruct((NACC,), _f32),
              jax.ShapeDtypeStruct((NACC,), _f32)],
    scratch_types=[
        pltpu.VMEM_SHARED((NACC,), _f32),
        pltpu.VMEM((NACC // 128, 128), _f32),
        pltpu.VMEM((2048,), _f32),
        pltpu.VMEM((33, 2, 128), _i32),
        pltpu.VMEM((C1,), _f32),
        pltpu.VMEM((33, 128), _f32),
        pltpu.VMEM((2, 16), _f32),
        pltpu.SemaphoreType.DMA,
    ],
    mesh=_mesh,
    compiler_params=_sc_params,
)
def _edge1(x_hbm, sd_hbm, a_hbm, wb_hbm,
           out0, out1, acc, x_v, z_v, e_c, a_f, m_c, wb_v, ss):
    c = lax.axis_index("c")
    s = lax.axis_index("s")
    wid = s * 2 + c
    pltpu.sync_copy(x_hbm, x_v)
    pltpu.sync_copy(wb_hbm, wb_v)
    z16 = jnp.zeros((16,), _f32)

    def zb(k, carry):
        z_v[pl.ds(k * 16, 16)] = z16
        return carry
    lax.fori_loop(0, 128, zb, 0)
    base = s * NTILE
    for t in range(3):
        pltpu.sync_copy(z_v, acc.at[pl.ds(base + t * 2048, 2048)])
    pltpu.sync_copy(z_v.at[pl.ds(0, 112)], acc.at[pl.ds(base + 3 * 2048, 112)])
    plsc.subcore_barrier()

    wv = wb_v[0]
    bv = wb_v[1]

    def chunk_body(g, carry):
        rb = wid * R1T + g * 33
        pltpu.sync_copy(sd_hbm.at[pl.ds(rb, 33)], e_c)
        pltpu.sync_copy(a_hbm.at[pl.ds(wid * (EPAD // 32) + g * C1, C1)], a_f)

        def row_body(j, carry2):
            for l in range(8):
                sv = e_c[j, 0, pl.ds(l * 16, 16)]
                av = a_f[pl.ds(j * 128 + l * 16, 16)]
                xg = plsc.load_gather(x_v, [sv >> 7, sv & 127])
                m_c[j, pl.ds(l * 16, 16)] = jnp.maximum(xg + av * wv + bv, 0.0)
            return carry2
        lax.fori_loop(0, 33, row_body, 0)
        hs = [pltpu.async_copy(m_c.at[j], acc.at[e_c.at[j, 1]], ss, add=True)
              for j in range(33)]
        for h in hs:
            h.wait()
        return carry
    lax.fori_loop(0, 12, chunk_body, 0)
    plsc.subcore_barrier()

    def wb(dst_hbm_ref):
        for t in range(3):
            sl = pl.ds(base + t * 2048, 2048)
            pltpu.sync_copy(acc.at[sl], z_v)
            pltpu.sync_copy(z_v, dst_hbm_ref.at[sl])
        sl = pl.ds(base + 3 * 2048, 112)
        pltpu.sync_copy(acc.at[sl], z_v.at[pl.ds(0, 112)])
        pltpu.sync_copy(z_v.at[pl.ds(0, 112)], dst_hbm_ref.at[sl])

    @pl.when(c == 0)
    def _():
        wb(out0)

    @pl.when(c == 1)
    def _():
        wb(out1)


@functools.partial(
    pl.kernel,
    out_type=[jax.ShapeDtypeStruct((NACC, 16), _f32),
              jax.ShapeDtypeStruct((NACC, 16), _f32)],
    scratch_types=[
        pltpu.VMEM_SHARED((NACC, 16), _f32),
        pltpu.VMEM((C2, 16), _f32),
        pltpu.VMEM((C2, 16), _f32),
        pltpu.VMEM((4, 2, 128), _i32),
        pltpu.VMEM((4, 2, 128), _i32),
        pltpu.VMEM((4, 2, 128), _i32),
        pltpu.VMEM((C2,), _f32),
        pltpu.VMEM((C2,), _f32),
        pltpu.VMEM((C2,), _f32),
        pltpu.VMEM((2, 2, 16), _f32),
        pltpu.SemaphoreType.DMA,
        pltpu.SemaphoreType.DMA,
        pltpu.SemaphoreType.DMA,
        pltpu.SemaphoreType.DMA,
        pltpu.SemaphoreType.DMA,
        pltpu.SemaphoreType.DMA,
        pltpu.SemaphoreType.DMA,
    ],
    mesh=_mesh,
    compiler_params=_sc_params,
)
def _edge23(hlo_hbm, hhi_hbm, sd_hbm, a_hbm, wb_hbm,
            out0, out1, acc, rows0, rows1, sd0, sd1, sd2, af0, af1, af2,
            wb_v, si0, si1, si2, sg0, sg1, ss0, ss1):
    c = lax.axis_index("c")
    s = lax.axis_index("s")
    rows = (rows0, rows1)
    sd = (sd0, sd1, sd2)
    af = (af0, af1, af2)
    si = (si0, si1, si2)
    sg = (sg0, sg1)
    ss = (ss0, ss1)
    pltpu.sync_copy(wb_hbm, wb_v)
    z16 = jnp.zeros((16,), _f32)

    def zb(k, carry):
        rows0[k] = z16
        return carry
    lax.fori_loop(0, C2, zb, 0)
    base = s * NTILE
    for t in range(12):
        pltpu.sync_copy(rows0, acc.at[pl.ds(base + t * C2, C2)])
    pltpu.sync_copy(rows0.at[pl.ds(0, 112)], acc.at[pl.ds(base + 12 * C2, 112)])
    plsc.subcore_barrier()

    wv = wb_v[c, 0]
    bv = wb_v[c, 1]
    rbase = s * (NC2 * 4)
    abase = s * (NC2 * C2)

    def load_idx(k, b):
        pltpu.async_copy(sd_hbm.at[pl.ds(rbase + k * 4, 4)], sd[b], si[b])
        pltpu.async_copy(a_hbm.at[pl.ds(abase + k * C2, C2)], af[b], si[b])

    def drain_idx(k, b):
        pltpu.make_async_copy(sd_hbm.at[pl.ds(rbase + k * 4, 4)],
                              sd[b], si[b]).wait()
        pltpu.make_async_copy(a_hbm.at[pl.ds(abase + k * C2, C2)],
                              af[b], si[b]).wait()

    def _gather_descs(rb, ib, sgb, tbl):
        return [pltpu.make_async_copy(tbl.at[sd[ib].at[j, 0]],
                                      rows[rb].at[pl.ds(j * 128, 128)],
                                      sg[sgb])
                for j in range(4)]

    def issue_gathers(rb, ib, sgb):
        @pl.when(c == 0)
        def _():
            for d in _gather_descs(rb, ib, sgb, hlo_hbm):
                d.start()

        @pl.when(c == 1)
        def _():
            for d in _gather_descs(rb, ib, sgb, hhi_hbm):
                d.start()

    def drain_gathers(sgb, rb, ib):
        @pl.when(c == 0)
        def _():
            for d in _gather_descs(rb, ib, sgb, hlo_hbm):
                d.wait()

        @pl.when(c == 1)
        def _():
            for d in _gather_descs(rb, ib, sgb, hhi_hbm):
                d.wait()

    def _scatter_descs(rb, ib, ssb):
        return [pltpu.make_async_copy(rows[rb].at[pl.ds(j * 128, 128)],
                                      acc.at[sd[ib].at[j, 1]], ss[ssb])
                for j in range(4)]

    def issue_scatters(rb, ib, ssb):
        for d in _scatter_descs(rb, ib, ssb):
            d.start(add=True)

    def drain_scatters(ssb, rb, ib):
        for d in _scatter_descs(rb, ib, ssb):
            d.wait()

    def compute(rb, ib):
        rk = rows[rb]
        ak = af[ib]

        def cb(q, carry):
            for uu in range(8):
                i = q * 8 + uu
                av = plsc.load_gather(ak, [jnp.full((16,), i, _i32)])
                rk[i] = jnp.maximum(rk[i] + av * wv + bv, 0.0)
            return carry
        lax.fori_loop(0, C2 // 8, cb, 0)

    load_idx(0, 0)
    load_idx(1, 1)
    drain_idx(0, 0)
    issue_gathers(0, 0, 0)

    def super_body(t, carry):
        for u in range(6):
            k = t * 6 + u
            ru, rn = u % 2, (u + 1) % 2
            iu, in1, in2 = u % 3, (u + 1) % 3, (u + 2) % 3
            if u == 0:
                @pl.when(k >= 1)
                def _():
                    drain_scatters(rn, rn, in2)
            else:
                drain_scatters(rn, rn, in2)

            @pl.when(k + 1 < NC2)
            def _():
                drain_idx(k + 1, in1)
                issue_gathers(rn, in1, rn)

            @pl.when(k + 2 < NC2)
            def _():
                load_idx(k + 2, in2)
            drain_gathers(ru, ru, iu)
            compute(ru, iu)
            issue_scatters(ru, iu, ru)
        return carry
    lax.fori_loop(0, NC2 // 6, super_body, 0)
    drain_scatters(1, 1, 2)
    plsc.subcore_barrier()

    def wb(dst_hbm_ref):
        for t in range(12):
            sl = pl.ds(base + t * C2, C2)
            pltpu.sync_copy(acc.at[sl], rows0)
            pltpu.sync_copy(rows0, dst_hbm_ref.at[sl])
        sl = pl.ds(base + 12 * C2, 112)
        pltpu.sync_copy(acc.at[sl], rows0.at[pl.ds(0, 112)])
        pltpu.sync_copy(rows0.at[pl.ds(0, 112)], dst_hbm_ref.at[sl])

    @pl.when(c == 0)
    def _():
        wb(out0)

    @pl.when(c == 1)
    def _():
        wb(out1)


R = 5000
G = N // R


def _leaky(z):
    return jnp.where(z >= 0, z, 0.01 * z)


def _full_spec(shape):
    return pl.BlockSpec(shape, lambda i: tuple(0 for _ in shape))


def _mlp1_body(x_ref, p0_ref, p1_ref, w1, b1, w2, b2, w3, b3, sc_ref,
               h_ref, st_ref):
    h0 = sc_ref[0, 0] * x_ref[...] + p0_ref[...] + p1_ref[...]
    z = _leaky(h0 * w1[...] + b1[...])
    z = _leaky(jnp.dot(z, w2[...], preferred_element_type=_f32) + b2[...])
    h = jnp.dot(z, w3[...], preferred_element_type=_f32) + b3[...]
    h_ref[...] = h
    st_ref[...] = jnp.stack([jnp.sum(h, 0), jnp.sum(h * h, 0)])[None]


def _mlp23_body(hlo_ref, hhi_ref, alo_ref, ahi_ref, w1, b1, w2, b2, w3, b3,
                sc_ref, h_ref, st_ref):
    h = jnp.concatenate([hlo_ref[...], hhi_ref[...]], axis=1)
    ag = jnp.concatenate([alo_ref[...], ahi_ref[...]], axis=1)
    z = sc_ref[0, 0] * h + ag
    z = _leaky(jnp.dot(z, w1[...], preferred_element_type=_f32) + b1[...])
    z = _leaky(jnp.dot(z, w2[...], preferred_element_type=_f32) + b2[...])
    h = jnp.dot(z, w3[...], preferred_element_type=_f32) + b3[...]
    h_ref[...] = h
    st_ref[...] = jnp.stack([jnp.sum(h, 0), jnp.sum(h * h, 0)])[None]


def _bn_split_body(h_ref, st_ref, g_ref, b_ref, lo_ref, hi_ref):
    st = st_ref[...]
    mu = jnp.sum(st[:, 0, :], 0) / N
    var = jnp.sum(st[:, 1, :], 0) / N - mu * mu
    y = g_ref[...] * (h_ref[...] - mu) * lax.rsqrt(var + 1e-5) + b_ref[...]
    y = jnp.maximum(y, 0.0)
    lo_ref[...] = y[:, :16]
    hi_ref[...] = y[:, 16:]


def _bn3_body(h_ref, st_ref, g_ref, b_ref, o_ref):
    st = st_ref[...]
    mu = jnp.sum(st[:, 0, :], 0) / N
    var = jnp.sum(st[:, 1, :], 0) / N - mu * mu
    o_ref[...] = g_ref[...] * (h_ref[...] - mu) * lax.rsqrt(var + 1e-5) + b_ref[...]


def _mlp1(x2, p0, p1, w1, b1, w2, b2, w3, b3, sc):
    return pl.pallas_call(
        _mlp1_body,
        grid=(G,),
        in_specs=[
            pl.BlockSpec((R, 1), lambda i: (i, 0)),
            pl.BlockSpec((R, 1), lambda i: (i, 0)),
            pl.BlockSpec((R, 1), lambda i: (i, 0)),
            _full_spec((1, 32)), _full_spec((1, 32)),
            _full_spec((32, 32)), _full_spec((1, 32)),
            _full_spec((32, 32)), _full_spec((1, 32)),
            _full_spec((1, 1)),
        ],
        out_specs=[
            pl.BlockSpec((R, 32), lambda i: (i, 0)),
            pl.BlockSpec((1, 2, 32), lambda i: (i, 0, 0)),
        ],
        out_shape=[
            jax.ShapeDtypeStruct((N, 32), _f32),
            jax.ShapeDtypeStruct((G, 2, 32), _f32),
        ],
    )(x2, p0, p1, w1, b1, w2, b2, w3, b3, sc)


def _mlp23(hlo, hhi, alo, ahi, w1, b1, w2, b2, w3, b3, sc, dout):
    return pl.pallas_call(
        _mlp23_body,
        grid=(G,),
        in_specs=[
            pl.BlockSpec((R, 16), lambda i: (i, 0)),
            pl.BlockSpec((R, 16), lambda i: (i, 0)),
            pl.BlockSpec((R, 16), lambda i: (i, 0)),
            pl.BlockSpec((R, 16), lambda i: (i, 0)),
            _full_spec((32, 32)), _full_spec((1, 32)),
            _full_spec((32, 32)), _full_spec((1, 32)),
            _full_spec((32, dout)), _full_spec((1, dout)),
            _full_spec((1, 1)),
        ],
        out_specs=[
            pl.BlockSpec((R, dout), lambda i: (i, 0)),
            pl.BlockSpec((1, 2, dout), lambda i: (i, 0, 0)),
        ],
        out_shape=[
            jax.ShapeDtypeStruct((N, dout), _f32),
            jax.ShapeDtypeStruct((G, 2, dout), _f32),
        ],
    )(hlo, hhi, alo, ahi, w1, b1, w2, b2, w3, b3, sc)


def _bn_split(h, st, g, b):
    return pl.pallas_call(
        _bn_split_body,
        grid=(G,),
        in_specs=[
            pl.BlockSpec((R, 32), lambda i: (i, 0)),
            _full_spec((G, 2, 32)),
            _full_spec((1, 32)), _full_spec((1, 32)),
        ],
        out_specs=[
            pl.BlockSpec((R, 16), lambda i: (i, 0)),
            pl.BlockSpec((R, 16), lambda i: (i, 0)),
        ],
        out_shape=[
            jax.ShapeDtypeStruct((N, 16), _f32),
            jax.ShapeDtypeStruct((N, 16), _f32),
        ],
    )(h, st, g, b)


def _bn3(h, st, g, b):
    return pl.pallas_call(
        _bn3_body,
        grid=(G,),
        in_specs=[
            pl.BlockSpec((R, 1), lambda i: (i, 0)),
            _full_spec((G, 2, 1)),
            _full_spec((1, 1)), _full_spec((1, 1)),
        ],
        out_specs=pl.BlockSpec((R, 1), lambda i: (i, 0)),
        out_shape=jax.ShapeDtypeStruct((N, 1), _f32),
    )(h, st, g, b)


def kernel(x, edge_index, edge_attr, params):
    src = edge_index[0]
    dst = edge_index[1]
    a = edge_attr[:, 0]
    pad_src = (jnp.arange(NPAD, dtype=_i32) * 911) % N
    src2 = jnp.concatenate([src, pad_src]).reshape(ROWS, 128)
    dst2 = jnp.concatenate([dst, jnp.full((NPAD,), N, _i32)]).reshape(ROWS, 128)
    sd = jnp.stack([src2, dst2], axis=1)
    af = jnp.concatenate([a, jnp.zeros((NPAD,), _f32)])

    p = params
    wb1 = jnp.stack([jnp.full((16,), p['We1'][0, 0]), jnp.full((16,), p['be1'][0])])
    wb2 = jnp.stack([p['We2'][0].reshape(2, 16), p['be2'].reshape(2, 16)], axis=1)
    wb3 = jnp.stack([p['We3'][0].reshape(2, 16), p['be3'].reshape(2, 16)], axis=1)

    def lin(layers, i):
        W, b = layers[i]
        return W, b.reshape(1, -1)

    sc1 = (1.0 + p['eps1']).reshape(1, 1)
    sc2 = (1.0 + p['eps2']).reshape(1, 1)
    sc3 = (1.0 + p['eps3']).reshape(1, 1)

    x2d = jnp.concatenate([x, jnp.zeros((NACC - N,), _f32)]).reshape(NACC // 128, 128)
    p0, p1 = _edge1(x2d, sd, af, wb1)
    w1, b1 = lin(p['nn1'], 0)
    w2, b2 = lin(p['nn1'], 1)
    w3, b3 = lin(p['nn1'], 2)
    h1, st1 = _mlp1(x.reshape(N, 1), p0.reshape(NACC, 1), p1.reshape(NACC, 1),
                    w1, b1, w2, b2, w3, b3, sc1)
    lo1, hi1 = _bn_split(h1, st1, p['bn1_g'].reshape(1, 32), p['bn1_b'].reshape(1, 32))

    alo2, ahi2 = _edge23(lo1, hi1, sd, af, wb2)
    w1, b1 = lin(p['nn2'], 0)
    w2, b2 = lin(p['nn2'], 1)
    w3, b3 = lin(p['nn2'], 2)
    h3, st2 = _mlp23(lo1, hi1, alo2, ahi2, w1, b1, w2, b2, w3, b3, sc2, 32)
    lo2, hi2 = _bn_split(h3, st2, p['bn2_g'].reshape(1, 32), p['bn2_b'].reshape(1, 32))

    alo3, ahi3 = _edge23(lo2, hi2, sd, af, wb3)
    w1, b1 = lin(p['nn3'], 0)
    w2, b2 = lin(p['nn3'], 1)
    w3, b3 = lin(p['nn3'], 2)
    h5, st3 = _mlp23(lo2, hi2, alo3, ahi3, w1, b1, w2, b2, w3, b3, sc3, 1)
    out = _bn3(h5, st3, p['bn3_g'].reshape(1, 1), p['bn3_b'].reshape(1, 1))
    return out


# speedup vs baseline: 19.5142x; 1.9136x over previous
import functools

import jax
import jax.numpy as jnp
from jax import lax
from jax.experimental import pallas as pl
from jax.experimental.pallas import tpu as pltpu
from jax.experimental.pallas import tpu_sc as plsc

N = 100000
E = 1600000
D = 32

EPAD = 1622016
NPAD = EPAD - E
ROWS = EPAD // 128
NACC = 100096
NTILE = NACC // 16

C1 = 4224
R1T = EPAD // 32 // 128
C2 = 512
NC2 = EPAD // 16 // C2

_mesh = plsc.VectorSubcoreMesh(core_axis_name="c", subcore_axis_name="s")
_sc_params = pltpu.CompilerParams(needs_layout_passes=False,
                                  use_tc_tiling_on_sc=False)

_i32 = jnp.int32
_f32 = jnp.float32


@functools.partial(
    pl.kernel,
    out_type=[jax.ShapeDtypeStruct((NACC,), _f32),
              jax.ShapeDtypeStruct((NACC,), _f32)],
    scratch_types=[
        pltpu.VMEM_SHARED((NACC,), _f32),
        pltpu.VMEM((NACC // 128, 128), _f32),
        pltpu.VMEM((2048,), _f32),
        pltpu.VMEM((33, 2, 128), _i32),
        pltpu.VMEM((C1,), _f32),
        pltpu.VMEM((33, 128), _f32),
        pltpu.VMEM((2, 16), _f32),
        pltpu.SemaphoreType.DMA,
    ],
    mesh=_mesh,
    compiler_params=_sc_params,
)
def _edge1(x_hbm, sd_hbm, a_hbm, wb_hbm,
           out0, out1, acc, x_v, z_v, e_c, a_f, m_c, wb_v, ss):
    c = lax.axis_index("c")
    s = lax.axis_index("s")
    wid = s * 2 + c
    pltpu.sync_copy(x_hbm, x_v)
    pltpu.sync_copy(wb_hbm, wb_v)
    z16 = jnp.zeros((16,), _f32)

    def zb(k, carry):
        z_v[pl.ds(k * 16, 16)] = z16
        return carry
    lax.fori_loop(0, 128, zb, 0)
    base = s * NTILE
    for t in range(3):
        pltpu.sync_copy(z_v, acc.at[pl.ds(base + t * 2048, 2048)])
    pltpu.sync_copy(z_v.at[pl.ds(0, 112)], acc.at[pl.ds(base + 3 * 2048, 112)])
    plsc.subcore_barrier()

    wv = wb_v[0]
    bv = wb_v[1]

    def chunk_body(g, carry):
        rb = wid * R1T + g * 33
        pltpu.sync_copy(sd_hbm.at[pl.ds(rb, 33)], e_c)
        pltpu.sync_copy(a_hbm.at[pl.ds(wid * (EPAD // 32) + g * C1, C1)], a_f)

        @plsc.parallel_loop(0, 33, 1, unroll=2)
        def _(j):
            for l in range(8):
                sv = e_c[j, 0, pl.ds(l * 16, 16)]
                av = a_f[pl.ds(j * 128 + l * 16, 16)]
                xg = plsc.load_gather(x_v, [sv >> 7, sv & 127])
                m_c[j, pl.ds(l * 16, 16)] = jnp.maximum(xg + av * wv + bv, 0.0)
        hs = [pltpu.async_copy(m_c.at[j], acc.at[e_c.at[j, 1]], ss, add=True)
              for j in range(33)]
        for h in hs:
            h.wait()
        return carry
    lax.fori_loop(0, 12, chunk_body, 0)
    plsc.subcore_barrier()

    def wb(dst_hbm_ref):
        for t in range(3):
            sl = pl.ds(base + t * 2048, 2048)
            pltpu.sync_copy(acc.at[sl], z_v)
            pltpu.sync_copy(z_v, dst_hbm_ref.at[sl])
        sl = pl.ds(base + 3 * 2048, 112)
        pltpu.sync_copy(acc.at[sl], z_v.at[pl.ds(0, 112)])
        pltpu.sync_copy(z_v.at[pl.ds(0, 112)], dst_hbm_ref.at[sl])

    @pl.when(c == 0)
    def _():
        wb(out0)

    @pl.when(c == 1)
    def _():
        wb(out1)


@functools.partial(
    pl.kernel,
    out_type=[jax.ShapeDtypeStruct((NACC, 16), _f32),
              jax.ShapeDtypeStruct((NACC, 16), _f32)],
    scratch_types=[
        pltpu.VMEM_SHARED((NACC, 16), _f32),
        pltpu.VMEM((C2, 16), _f32),
        pltpu.VMEM((C2, 16), _f32),
        pltpu.VMEM((4, 2, 128), _i32),
        pltpu.VMEM((4, 2, 128), _i32),
        pltpu.VMEM((4, 2, 128), _i32),
        pltpu.VMEM((C2,), _f32),
        pltpu.VMEM((C2,), _f32),
        pltpu.VMEM((C2,), _f32),
        pltpu.VMEM((2, 2, 16), _f32),
        pltpu.SemaphoreType.DMA,
        pltpu.SemaphoreType.DMA,
        pltpu.SemaphoreType.DMA,
        pltpu.SemaphoreType.DMA,
        pltpu.SemaphoreType.DMA,
        pltpu.SemaphoreType.DMA,
        pltpu.SemaphoreType.DMA,
    ],
    mesh=_mesh,
    compiler_params=_sc_params,
)
def _edge23(hlo_hbm, hhi_hbm, sd_hbm, a_hbm, wb_hbm,
            out0, out1, acc, rows0, rows1, sd0, sd1, sd2, af0, af1, af2,
            wb_v, si0, si1, si2, sg0, sg1, ss0, ss1):
    c = lax.axis_index("c")
    s = lax.axis_index("s")
    rows = (rows0, rows1)
    sd = (sd0, sd1, sd2)
    af = (af0, af1, af2)
    si = (si0, si1, si2)
    sg = (sg0, sg1)
    ss = (ss0, ss1)
    pltpu.sync_copy(wb_hbm, wb_v)
    z16 = jnp.zeros((16,), _f32)

    def zb(k, carry):
        rows0[k] = z16
        return carry
    lax.fori_loop(0, C2, zb, 0)
    base = s * NTILE
    for t in range(12):
        pltpu.sync_copy(rows0, acc.at[pl.ds(base + t * C2, C2)])
    pltpu.sync_copy(rows0.at[pl.ds(0, 112)], acc.at[pl.ds(base + 12 * C2, 112)])
    plsc.subcore_barrier()

    wv = wb_v[c, 0]
    bv = wb_v[c, 1]
    rbase = s * (NC2 * 4)
    abase = s * (NC2 * C2)

    def load_idx(k, b):
        pltpu.async_copy(sd_hbm.at[pl.ds(rbase + k * 4, 4)], sd[b], si[b])
        pltpu.async_copy(a_hbm.at[pl.ds(abase + k * C2, C2)], af[b], si[b])

    def drain_idx(k, b):
        pltpu.make_async_copy(sd_hbm.at[pl.ds(rbase + k * 4, 4)],
                              sd[b], si[b]).wait()
        pltpu.make_async_copy(a_hbm.at[pl.ds(abase + k * C2, C2)],
                              af[b], si[b]).wait()

    def _gather_descs(rb, ib, sgb, tbl):
        return [pltpu.make_async_copy(tbl.at[sd[ib].at[j, 0]],
                                      rows[rb].at[pl.ds(j * 128, 128)],
                                      sg[sgb])
                for j in range(4)]

    def issue_gathers(rb, ib, sgb):
        @pl.when(c == 0)
        def _():
            for d in _gather_descs(rb, ib, sgb, hlo_hbm):
                d.start()

        @pl.when(c == 1)
        def _():
            for d in _gather_descs(rb, ib, sgb, hhi_hbm):
                d.start()

    def drain_gathers(sgb, rb, ib):
        @pl.when(c == 0)
        def _():
            for d in _gather_descs(rb, ib, sgb, hlo_hbm):
                d.wait()

        @pl.when(c == 1)
        def _():
            for d in _gather_descs(rb, ib, sgb, hhi_hbm):
                d.wait()

    def _scatter_descs(rb, ib, ssb):
        return [pltpu.make_async_copy(rows[rb].at[pl.ds(j * 128, 128)],
                                      acc.at[sd[ib].at[j, 1]], ss[ssb])
                for j in range(4)]

    def issue_scatters(rb, ib, ssb):
        for d in _scatter_descs(rb, ib, ssb):
            d.start(add=True)

    def drain_scatters(ssb, rb, ib):
        for d in _scatter_descs(rb, ib, ssb):
            d.wait()

    def compute(rb, ib):
        rk = rows[rb]
        ak = af[ib]

        @plsc.parallel_loop(0, C2, 1, unroll=8, carry=jnp.zeros((16,), _i32))
        def _(i, iv):
            av = plsc.load_gather(ak, [iv])
            rk[i] = jnp.maximum(rk[i] + av * wv + bv, 0.0)
            return iv + 1

    load_idx(0, 0)
    load_idx(1, 1)
    drain_idx(0, 0)
    issue_gathers(0, 0, 0)

    def super_body(t, carry):
        for u in range(6):
            k = t * 6 + u
            ru, rn = u % 2, (u + 1) % 2
            iu, in1, in2 = u % 3, (u + 1) % 3, (u + 2) % 3
            if u == 0:
                @pl.when(k >= 1)
                def _():
                    drain_scatters(rn, rn, in2)
            else:
                drain_scatters(rn, rn, in2)

            @pl.when(k + 1 < NC2)
            def _():
                drain_idx(k + 1, in1)
                issue_gathers(rn, in1, rn)

            @pl.when(k + 2 < NC2)
            def _():
                load_idx(k + 2, in2)
            drain_gathers(ru, ru, iu)
            compute(ru, iu)
            issue_scatters(ru, iu, ru)
        return carry
    lax.fori_loop(0, NC2 // 6, super_body, 0)
    drain_scatters(1, 1, 2)
    plsc.subcore_barrier()

    def wb(dst_hbm_ref):
        for t in range(12):
            sl = pl.ds(base + t * C2, C2)
            pltpu.sync_copy(acc.at[sl], rows0)
            pltpu.sync_copy(rows0, dst_hbm_ref.at[sl])
        sl = pl.ds(base + 12 * C2, 112)
        pltpu.sync_copy(acc.at[sl], rows0.at[pl.ds(0, 112)])
        pltpu.sync_copy(rows0.at[pl.ds(0, 112)], dst_hbm_ref.at[sl])

    @pl.when(c == 0)
    def _():
        wb(out0)

    @pl.when(c == 1)
    def _():
        wb(out1)


R = 5000
G = N // R


def _leaky(z):
    return jnp.where(z >= 0, z, 0.01 * z)


def _full_spec(shape):
    return pl.BlockSpec(shape, lambda i: tuple(0 for _ in shape))


def _mlp1_body(x_ref, p0_ref, p1_ref, w1, b1, w2, b2, w3, b3, sc_ref,
               h_ref, st_ref):
    h0 = sc_ref[0, 0] * x_ref[...] + p0_ref[...] + p1_ref[...]
    z = _leaky(h0 * w1[...] + b1[...])
    z = _leaky(jnp.dot(z, w2[...], preferred_element_type=_f32) + b2[...])
    h = jnp.dot(z, w3[...], preferred_element_type=_f32) + b3[...]
    h_ref[...] = h
    st_ref[...] = jnp.stack([jnp.sum(h, 0), jnp.sum(h * h, 0)])[None]


def _mlp23_body(hlo_ref, hhi_ref, alo_ref, ahi_ref, w1, b1, w2, b2, w3, b3,
                sc_ref, h_ref, st_ref):
    h = jnp.concatenate([hlo_ref[...], hhi_ref[...]], axis=1)
    ag = jnp.concatenate([alo_ref[...], ahi_ref[...]], axis=1)
    z = sc_ref[0, 0] * h + ag
    z = _leaky(jnp.dot(z, w1[...], preferred_element_type=_f32) + b1[...])
    z = _leaky(jnp.dot(z, w2[...], preferred_element_type=_f32) + b2[...])
    h = jnp.dot(z, w3[...], preferred_element_type=_f32) + b3[...]
    h_ref[...] = h
    st_ref[...] = jnp.stack([jnp.sum(h, 0), jnp.sum(h * h, 0)])[None]


def _bn_split_body(h_ref, st_ref, g_ref, b_ref, lo_ref, hi_ref):
    st = st_ref[...]
    mu = jnp.sum(st[:, 0, :], 0) / N
    var = jnp.sum(st[:, 1, :], 0) / N - mu * mu
    y = g_ref[...] * (h_ref[...] - mu) * lax.rsqrt(var + 1e-5) + b_ref[...]
    y = jnp.maximum(y, 0.0)
    lo_ref[...] = y[:, :16]
    hi_ref[...] = y[:, 16:]


def _bn3_body(h_ref, st_ref, g_ref, b_ref, o_ref):
    st = st_ref[...]
    mu = jnp.sum(st[:, 0, :], 0) / N
    var = jnp.sum(st[:, 1, :], 0) / N - mu * mu
    o_ref[...] = g_ref[...] * (h_ref[...] - mu) * lax.rsqrt(var + 1e-5) + b_ref[...]


def _mlp1(x2, p0, p1, w1, b1, w2, b2, w3, b3, sc):
    return pl.pallas_call(
        _mlp1_body,
        grid=(G,),
        in_specs=[
            pl.BlockSpec((R, 1), lambda i: (i, 0)),
            pl.BlockSpec((R, 1), lambda i: (i, 0)),
            pl.BlockSpec((R, 1), lambda i: (i, 0)),
            _full_spec((1, 32)), _full_spec((1, 32)),
            _full_spec((32, 32)), _full_spec((1, 32)),
            _full_spec((32, 32)), _full_spec((1, 32)),
            _full_spec((1, 1)),
        ],
        out_specs=[
            pl.BlockSpec((R, 32), lambda i: (i, 0)),
            pl.BlockSpec((1, 2, 32), lambda i: (i, 0, 0)),
        ],
        out_shape=[
            jax.ShapeDtypeStruct((N, 32), _f32),
            jax.ShapeDtypeStruct((G, 2, 32), _f32),
        ],
    )(x2, p0, p1, w1, b1, w2, b2, w3, b3, sc)


def _mlp23(hlo, hhi, alo, ahi, w1, b1, w2, b2, w3, b3, sc, dout):
    return pl.pallas_call(
        _mlp23_body,
        grid=(G,),
        in_specs=[
            pl.BlockSpec((R, 16), lambda i: (i, 0)),
            pl.BlockSpec((R, 16), lambda i: (i, 0)),
            pl.BlockSpec((R, 16), lambda i: (i, 0)),
            pl.BlockSpec((R, 16), lambda i: (i, 0)),
            _full_spec((32, 32)), _full_spec((1, 32)),
            _full_spec((32, 32)), _full_spec((1, 32)),
            _full_spec((32, dout)), _full_spec((1, dout)),
            _full_spec((1, 1)),
        ],
        out_specs=[
            pl.BlockSpec((R, dout), lambda i: (i, 0)),
            pl.BlockSpec((1, 2, dout), lambda i: (i, 0, 0)),
        ],
        out_shape=[
            jax.ShapeDtypeStruct((N, dout), _f32),
            jax.ShapeDtypeStruct((G, 2, dout), _f32),
        ],
    )(hlo, hhi, alo, ahi, w1, b1, w2, b2, w3, b3, sc)


def _bn_split(h, st, g, b):
    return pl.pallas_call(
        _bn_split_body,
        grid=(G,),
        in_specs=[
            pl.BlockSpec((R, 32), lambda i: (i, 0)),
            _full_spec((G, 2, 32)),
            _full_spec((1, 32)), _full_spec((1, 32)),
        ],
        out_specs=[
            pl.BlockSpec((R, 16), lambda i: (i, 0)),
            pl.BlockSpec((R, 16), lambda i: (i, 0)),
        ],
        out_shape=[
            jax.ShapeDtypeStruct((N, 16), _f32),
            jax.ShapeDtypeStruct((N, 16), _f32),
        ],
    )(h, st, g, b)


def _bn3(h, st, g, b):
    return pl.pallas_call(
        _bn3_body,
        grid=(G,),
        in_specs=[
            pl.BlockSpec((R, 1), lambda i: (i, 0)),
            _full_spec((G, 2, 1)),
            _full_spec((1, 1)), _full_spec((1, 1)),
        ],
        out_specs=pl.BlockSpec((R, 1), lambda i: (i, 0)),
        out_shape=jax.ShapeDtypeStruct((N, 1), _f32),
    )(h, st, g, b)


def kernel(x, edge_index, edge_attr, params):
    src = edge_index[0]
    dst = edge_index[1]
    a = edge_attr[:, 0]
    pad_src = (jnp.arange(NPAD, dtype=_i32) * 911) % N
    src2 = jnp.concatenate([src, pad_src]).reshape(ROWS, 128)
    dst2 = jnp.concatenate([dst, jnp.full((NPAD,), N, _i32)]).reshape(ROWS, 128)
    sd = jnp.stack([src2, dst2], axis=1)
    af = jnp.concatenate([a, jnp.zeros((NPAD,), _f32)])

    p = params
    wb1 = jnp.stack([jnp.full((16,), p['We1'][0, 0]), jnp.full((16,), p['be1'][0])])
    wb2 = jnp.stack([p['We2'][0].reshape(2, 16), p['be2'].reshape(2, 16)], axis=1)
    wb3 = jnp.stack([p['We3'][0].reshape(2, 16), p['be3'].reshape(2, 16)], axis=1)

    def lin(layers, i):
        W, b = layers[i]
        return W, b.reshape(1, -1)

    sc1 = (1.0 + p['eps1']).reshape(1, 1)
    sc2 = (1.0 + p['eps2']).reshape(1, 1)
    sc3 = (1.0 + p['eps3']).reshape(1, 1)

    x2d = jnp.concatenate([x, jnp.zeros((NACC - N,), _f32)]).reshape(NACC // 128, 128)
    p0, p1 = _edge1(x2d, sd, af, wb1)
    w1, b1 = lin(p['nn1'], 0)
    w2, b2 = lin(p['nn1'], 1)
    w3, b3 = lin(p['nn1'], 2)
    h1, st1 = _mlp1(x.reshape(N, 1), p0.reshape(NACC, 1), p1.reshape(NACC, 1),
                    w1, b1, w2, b2, w3, b3, sc1)
    lo1, hi1 = _bn_split(h1, st1, p['bn1_g'].reshape(1, 32), p['bn1_b'].reshape(1, 32))

    alo2, ahi2 = _edge23(lo1, hi1, sd, af, wb2)
    w1, b1 = lin(p['nn2'], 0)
    w2, b2 = lin(p['nn2'], 1)
    w3, b3 = lin(p['nn2'], 2)
    h3, st2 = _mlp23(lo1, hi1, alo2, ahi2, w1, b1, w2, b2, w3, b3, sc2, 32)
    lo2, hi2 = _bn_split(h3, st2, p['bn2_g'].reshape(1, 32), p['bn2_b'].reshape(1, 32))

    alo3, ahi3 = _edge23(lo2, hi2, sd, af, wb3)
    w1, b1 = lin(p['nn3'], 0)
    w2, b2 = lin(p['nn3'], 1)
    w3, b3 = lin(p['nn3'], 2)
    h5, st3 = _mlp23(lo2, hi2, alo3, ahi3, w1, b1, w2, b2, w3, b3, sc3, 1)
    out = _bn3(h5, st3, p['bn3_g'].reshape(1, 1), p['bn3_b'].reshape(1, 1))
    return out
